# Initial kernel scaffold; baseline (speedup 1.0000x reference)
import jax
import jax.numpy as jnp
from jax.experimental import pallas as pl


def kernel(x, edge_index, W1, a_src1, a_dst1, b1, W2, a_src2, a_dst2, b2, W3, a_src3, a_dst3, b3):
    raise NotImplementedError("write your pallas kernel here")


# speedup vs baseline: 37.6731x; 37.6731x over previous
import functools

import jax
import jax.numpy as jnp
from jax import lax
from jax.experimental import pallas as pl
from jax.experimental.pallas import tpu as pltpu
from jax.experimental.pallas import tpu_sc as plsc

_N = 10000
_E = 320000
_NC = 2
_NS = 16
_NW = _NC * _NS
_EPW = _E // _NW
_K = 400
_NCH = _EPW // _K
_AW = 80
_BN = 1000
_GRID = _N // _BN
_NP = 10240
_ZR = _NP // _NS
_f32 = jnp.float32

_GDN = jax.lax.GatherDimensionNumbers(
    offset_dims=(), collapsed_slice_dims=(0,), start_index_map=(0,))


def _bcast_lane(v, lane):
    idx = jnp.full((16,), lane, jnp.int32)
    return jax.lax.gather(v, idx[:, None], _GDN, slice_sizes=(1,),
                          mode=jax.lax.GatherScatterMode.PROMISE_IN_BOUNDS)


def _make_sc_layer(nph, nheads, bph, shared_adq):
    mesh = plsc.VectorSubcoreMesh(core_axis_name="c", subcore_axis_name="s")
    nadq = 1 if shared_adq else nph

    @functools.partial(
        pl.kernel,
        out_type=jax.ShapeDtypeStruct((nph * 2 * _NP, _AW), _f32),
        mesh=mesh,
        compiler_params=pltpu.CompilerParams(use_tc_tiling_on_sc=False),
        scratch_types=[
            pltpu.VMEM((_K,), jnp.int32),
            pltpu.VMEM((_K,), jnp.int32),
            pltpu.VMEM((_K, _AW), _f32),
            pltpu.VMEM((_K, 16), _f32),
            pltpu.VMEM_SHARED((_NP, _AW), _f32),
            pltpu.SemaphoreType.DMA,
            pltpu.SemaphoreType.DMA,
        ],
    )
    def kern(*refs):
        htabs = refs[:nph]
        adqs = refs[nph:nph + nadq]
        src, dst, out = refs[nph + nadq:nph + nadq + 3]
        src_v, dst_v, hrows, adrows, acc, sem1, sem2 = refs[nph + nadq + 3:]
        cid = lax.axis_index("c")
        sid = lax.axis_index("s")
        wid = sid * _NC + cid
        ebase = wid * _EPW
        r0 = sid * _ZR

        zero = jnp.zeros((16,), _f32)

        for p in range(nph):
            htab = htabs[p]
            adq = adqs[0 if shared_adq else p]

            def zbody(r, carry):
                for cb in range(_AW // 16):
                    hrows[r, pl.ds(cb * 16, 16)] = zero
                return carry

            lax.fori_loop(0, 128, zbody, None)
            for j in range(_ZR // 128):
                pltpu.sync_copy(hrows.at[pl.ds(0, 128)],
                                acc.at[pl.ds(r0 + j * 128, 128)])
            plsc.subcore_barrier()

            def chunk_body(i, carry):
                base = ebase + i * _K
                pltpu.sync_copy(src.at[pl.ds(base, _K)], src_v)
                pltpu.sync_copy(dst.at[pl.ds(base, _K)], dst_v)
                cp1 = pltpu.async_copy(htab.at[src_v], hrows, sem1)
                cp2 = pltpu.async_copy(adq.at[dst_v], adrows, sem2)
                cp1.wait()
                cp2.wait()

                def edge_body(e, ecarry):
                    asv = hrows[e, pl.ds(64, 16)]
                    adv = adrows[e, pl.ds(0, 16)]
                    ev = asv + adv
                    ev = jnp.where(ev > 0, ev, ev * 0.2)
                    exv = jnp.exp(ev)
                    hrows[e, pl.ds(64, 16)] = exv
                    for h in range(nheads):
                        sv = _bcast_lane(exv, h)
                        for j in range(bph):
                            cb = h * bph + j
                            hv = hrows[e, pl.ds(cb * 16, 16)]
                            hrows[e, pl.ds(cb * 16, 16)] = hv * sv
                    return ecarry

                lax.fori_loop(0, _K, edge_body, None)
                pltpu.sync_copy(hrows, acc.at[dst_v], add=True)
                return carry

            lax.fori_loop(0, _NCH, chunk_body, None)
            plsc.subcore_barrier()

            obase = (p * 2 + cid) * _NP + r0
            for j in range(_ZR // 128):
                pltpu.sync_copy(acc.at[pl.ds(r0 + j * 128, 128)],
                                out.at[pl.ds(obase + j * 128, 128)])

    return kern


_sc_l1 = _make_sc_layer(4, 2, 2, False)
_sc_l2 = _make_sc_layer(2, 4, 1, False)
_sc_l3 = _make_sc_layer(2, 1, 4, True)


def _a1_body(x_ref, w1_ref, asm_ref, adm_ref,
             ht0, ht1, ht2, ht3, ad0, ad1, ad2, ad3, exs_ref):
    h = jnp.dot(x_ref[...], w1_ref[...], preferred_element_type=_f32)
    als = jnp.dot(h, asm_ref[...], preferred_element_type=_f32)
    ald = jnp.dot(h, adm_ref[...], preferred_element_type=_f32)
    z14 = jnp.zeros((_BN, 14), _f32)
    for p, ht in enumerate((ht0, ht1, ht2, ht3)):
        ht[...] = jnp.concatenate(
            [h[:, 64 * p:64 * p + 64], als[:, 2 * p:2 * p + 2], z14], axis=1)
    for p, ad in enumerate((ad0, ad1, ad2, ad3)):
        ad[...] = jnp.concatenate([ald[:, 2 * p:2 * p + 2], z14], axis=1)
    es = als + ald
    es = jnp.where(es > 0, es, es * 0.2)
    exs_ref[...] = jnp.exp(es)


def _n1_body(acc_ref, ht0_ref, ht1_ref, ht2_ref, ht3_ref, exs_ref, b1_ref,
             w2_ref, as2_ref, ad2_ref, r1_ref,
             o_ht0, o_ht1, o_ad0, o_ad1, exs2_ref):
    a = [acc_ref[2 * p] + acc_ref[2 * p + 1] for p in range(4)]
    exs = exs_ref[...]
    h1 = jnp.concatenate(
        [r[:, :64] for r in (ht0_ref, ht1_ref, ht2_ref, ht3_ref)], axis=1)
    num = jnp.concatenate([ap[:, :64] for ap in a], axis=1)
    den8 = jnp.concatenate([ap[:, 64:66] for ap in a], axis=1) + exs
    exb = jnp.dot(exs, r1_ref[...], preferred_element_type=_f32)
    denb = jnp.dot(den8, r1_ref[...], preferred_element_type=_f32)
    out1 = (num + exb * h1) / (denb + 1e-16) + b1_ref[...]
    h1r = jnp.maximum(out1, 0.0)
    h2 = jnp.dot(h1r, w2_ref[...], preferred_element_type=_f32)
    as2 = jnp.dot(h2, as2_ref[...], preferred_element_type=_f32)
    ad2 = jnp.dot(h2, ad2_ref[...], preferred_element_type=_f32)
    z12 = jnp.zeros((_BN, 12), _f32)
    for p, o in enumerate((o_ht0, o_ht1)):
        o[...] = jnp.concatenate(
            [h2[:, 64 * p:64 * p + 64], as2[:, 4 * p:4 * p + 4], z12], axis=1)
    for p, o in enumerate((o_ad0, o_ad1)):
        o[...] = jnp.concatenate([ad2[:, 4 * p:4 * p + 4], z12], axis=1)
    es = as2 + ad2
    es = jnp.where(es > 0, es, es * 0.2)
    exs2_ref[...] = jnp.exp(es)


def _n2_body(acc_ref, ht0_ref, ht1_ref, exs2_ref, b2_ref, w3_ref,
             as3_ref, ad3_ref, r2_ref,
             o_ht0, o_ht1, o_adq, exs3_ref):
    a = [acc_ref[2 * p] + acc_ref[2 * p + 1] for p in range(2)]
    exs = exs2_ref[...]
    h2 = jnp.concatenate([ht0_ref[:, :64], ht1_ref[:, :64]], axis=1)
    num = jnp.concatenate([ap[:, :64] for ap in a], axis=1)
    den8 = jnp.concatenate([ap[:, 64:68] for ap in a], axis=1) + exs
    exb = jnp.dot(exs, r2_ref[...], preferred_element_type=_f32)
    denb = jnp.dot(den8, r2_ref[...], preferred_element_type=_f32)
    out2 = (num + exb * h2) / (denb + 1e-16) + b2_ref[...]
    h2r = jnp.maximum(out2, 0.0)
    h3 = jnp.dot(h2r, w3_ref[...], preferred_element_type=_f32)
    as3 = jnp.sum(h3 * as3_ref[...], axis=1, keepdims=True)
    ad3 = jnp.sum(h3 * ad3_ref[...], axis=1, keepdims=True)
    z15 = jnp.zeros((_BN, 15), _f32)
    for p, o in enumerate((o_ht0, o_ht1)):
        o[...] = jnp.concatenate(
            [h3[:, 64 * p:64 * p + 64], as3, z15], axis=1)
    o_adq[...] = jnp.concatenate([ad3, z15], axis=1)
    es = as3 + ad3
    es = jnp.where(es > 0, es, es * 0.2)
    exs3_ref[...] = jnp.concatenate([jnp.exp(es), jnp.zeros((_BN, 7), _f32)],
                                    axis=1)


def _n3_body(acc_ref, ht0_ref, ht1_ref, exs3_ref, b3_ref, out_ref):
    a0 = acc_ref[0] + acc_ref[1]
    a1 = acc_ref[2] + acc_ref[3]
    ex3 = exs3_ref[:, 0:1]
    h3 = jnp.concatenate([ht0_ref[:, :64], ht1_ref[:, :64]], axis=1)
    num = jnp.concatenate([a0[:, :64], a1[:, :64]], axis=1)
    den = a0[:, 64:65] + ex3 + 1e-16
    out_ref[...] = (num + ex3 * h3) / den + b3_ref[...]


def _row_block(i):
    return (i, 0)


def _full_block(i):
    return (0, 0)


def _acc_block(i):
    return (0, i, 0)


def _ht_spec():
    return pl.BlockSpec((_BN, _AW), _row_block)


def _adq_spec():
    return pl.BlockSpec((_BN, 16), _row_block)


def _ht_shape():
    return jax.ShapeDtypeStruct((_N, _AW), _f32)


def _adq_shape():
    return jax.ShapeDtypeStruct((_N, 16), _f32)


def kernel(x, edge_index, W1, a_src1, a_dst1, b1, W2, a_src2, a_dst2, b2,
           W3, a_src3, a_dst3, b3):
    src = edge_index[0]
    dst = edge_index[1]
    eye8 = jnp.eye(8, dtype=_f32)
    AS1 = jnp.einsum("hk,hc->hck", eye8, a_src1).reshape(256, 8)
    AD1 = jnp.einsum("hk,hc->hck", eye8, a_dst1).reshape(256, 8)
    AS2 = jnp.einsum("hk,hc->hck", eye8, a_src2).reshape(128, 8)
    AD2 = jnp.einsum("hk,hc->hck", eye8, a_dst2).reshape(128, 8)
    R1 = jnp.kron(eye8, jnp.ones((1, 32), _f32))
    R2 = jnp.kron(eye8, jnp.ones((1, 16), _f32))

    l1 = pl.pallas_call(
        _a1_body,
        grid=(_GRID,),
        in_specs=[
            pl.BlockSpec((_BN, 128), _row_block),
            pl.BlockSpec((128, 256), _full_block),
            pl.BlockSpec((256, 8), _full_block),
            pl.BlockSpec((256, 8), _full_block),
        ],
        out_specs=[_ht_spec() for _ in range(4)]
        + [_adq_spec() for _ in range(4)]
        + [pl.BlockSpec((_BN, 8), _row_block)],
        out_shape=[_ht_shape() for _ in range(4)]
        + [_adq_shape() for _ in range(4)]
        + [jax.ShapeDtypeStruct((_N, 8), _f32)],
    )(x, W1, AS1, AD1)
    hts1, adqs1, exs1 = l1[:4], l1[4:8], l1[8]

    acc1 = _sc_l1(*hts1, *adqs1, src, dst).reshape(8, _NP, _AW)

    ht2_0, ht2_1, ad2_0, ad2_1, exs2 = pl.pallas_call(
        _n1_body,
        grid=(_GRID,),
        in_specs=[
            pl.BlockSpec((8, _BN, _AW), _acc_block),
            _ht_spec(), _ht_spec(), _ht_spec(), _ht_spec(),
            pl.BlockSpec((_BN, 8), _row_block),
            pl.BlockSpec((1, 256), _full_block),
            pl.BlockSpec((256, 128), _full_block),
            pl.BlockSpec((128, 8), _full_block),
            pl.BlockSpec((128, 8), _full_block),
            pl.BlockSpec((8, 256), _full_block),
        ],
        out_specs=[_ht_spec(), _ht_spec(), _adq_spec(), _adq_spec(),
                   pl.BlockSpec((_BN, 8), _row_block)],
        out_shape=[_ht_shape(), _ht_shape(), _adq_shape(), _adq_shape(),
                   jax.ShapeDtypeStruct((_N, 8), _f32)],
    )(acc1, *hts1, exs1, b1.reshape(1, 256), W2, AS2, AD2, R1)

    acc2 = _sc_l2(ht2_0, ht2_1, ad2_0, ad2_1, src, dst).reshape(4, _NP, _AW)

    ht3_0, ht3_1, adq3, exs3 = pl.pallas_call(
        _n2_body,
        grid=(_GRID,),
        in_specs=[
            pl.BlockSpec((4, _BN, _AW), _acc_block),
            _ht_spec(), _ht_spec(),
            pl.BlockSpec((_BN, 8), _row_block),
            pl.BlockSpec((1, 128), _full_block),
            pl.BlockSpec((128, 128), _full_block),
            pl.BlockSpec((1, 128), _full_block),
            pl.BlockSpec((1, 128), _full_block),
            pl.BlockSpec((8, 128), _full_block),
        ],
        out_specs=[_ht_spec(), _ht_spec(), _adq_spec(),
                   pl.BlockSpec((_BN, 8), _row_block)],
        out_shape=[_ht_shape(), _ht_shape(), _adq_shape(),
                   jax.ShapeDtypeStruct((_N, 8), _f32)],
    )(acc2, ht2_0, ht2_1, exs2, b2.reshape(1, 128), W3,
      a_src3.reshape(1, 128), a_dst3.reshape(1, 128), R2)

    acc3 = _sc_l3(ht3_0, ht3_1, adq3, src, dst).reshape(4, _NP, _AW)

    out = pl.pallas_call(
        _n3_body,
        grid=(_GRID,),
        in_specs=[
            pl.BlockSpec((4, _BN, _AW), _acc_block),
            _ht_spec(), _ht_spec(),
            pl.BlockSpec((_BN, 8), _row_block),
            pl.BlockSpec((1, 128), _full_block),
        ],
        out_specs=pl.BlockSpec((_BN, 128), _row_block),
        out_shape=jax.ShapeDtypeStruct((_N, 128), _f32),
    )(acc3, ht3_0, ht3_1, exs3, b3.reshape(1, 128))

    return out


# speedup vs baseline: 48.5535x; 1.2888x over previous
import functools

import jax
import jax.numpy as jnp
from jax import lax
from jax.experimental import pallas as pl
from jax.experimental.pallas import tpu as pltpu
from jax.experimental.pallas import tpu_sc as plsc

_N = 10000
_E = 320000
_NC = 2
_NS = 16
_NW = _NC * _NS
_EPW = _E // _NW
_K = 125
_NCH = _EPW // _K
_NPAIR = _NCH // 2
_AW = 80
_BN = 1000
_GRID = _N // _BN
_NP = 10240
_ZR = _NP // _NS
_f32 = jnp.float32

_GDN = jax.lax.GatherDimensionNumbers(
    offset_dims=(), collapsed_slice_dims=(0,), start_index_map=(0,))


def _bcast_lane(v, lane):
    idx = jnp.full((16,), lane, jnp.int32)
    return jax.lax.gather(v, idx[:, None], _GDN, slice_sizes=(1,),
                          mode=jax.lax.GatherScatterMode.PROMISE_IN_BOUNDS)


def _make_sc_layer(nph, nheads, bph, shared_adq):
    mesh = plsc.VectorSubcoreMesh(core_axis_name="c", subcore_axis_name="s")
    nadq = 1 if shared_adq else nph

    @functools.partial(
        pl.kernel,
        out_type=jax.ShapeDtypeStruct((nph * 2 * _NP, _AW), _f32),
        mesh=mesh,
        compiler_params=pltpu.CompilerParams(use_tc_tiling_on_sc=False),
        scratch_types=[
            pltpu.VMEM((_NCH, _K), jnp.int32),
            pltpu.VMEM((_NCH, _K), jnp.int32),
            pltpu.VMEM((128, _AW), _f32),
            pltpu.VMEM((128, _AW), _f32),
            pltpu.VMEM((_K, 16), _f32),
            pltpu.VMEM((_K, 16), _f32),
            pltpu.VMEM_SHARED((_NP, _AW), _f32),
            pltpu.SemaphoreType.DMA,
            pltpu.SemaphoreType.DMA,
            pltpu.SemaphoreType.DMA,
            pltpu.SemaphoreType.DMA,
            pltpu.SemaphoreType.DMA,
            pltpu.SemaphoreType.DMA,
        ],
    )
    def kern(*refs):
        htabs = refs[:nph]
        adqs = refs[nph:nph + nadq]
        src, dst, out = refs[nph + nadq:nph + nadq + 3]
        (src_all, dst_all, hrow0, hrow1, adrow0, adrow1, acc,
         sgh0, sga0, ss0, sgh1, sga1, ss1) = refs[nph + nadq + 3:]
        cid = lax.axis_index("c")
        sid = lax.axis_index("s")
        wid = sid * _NC + cid
        r0 = sid * _ZR

        zero = jnp.zeros((16,), _f32)
        hbufs = (hrow0, hrow1)
        abufs = (adrow0, adrow1)
        ghsems = (sgh0, sgh1)
        gasems = (sga0, sga1)
        ssems = (ss0, ss1)

        pltpu.sync_copy(src.at[wid], src_all)
        pltpu.sync_copy(dst.at[wid], dst_all)

        def compute_chunk(hbuf, abuf):
            def edge_body(e, ecarry):
                asv = hbuf[e, pl.ds(64, 16)]
                adv = abuf[e, pl.ds(0, 16)]
                ev = asv + adv
                ev = jnp.where(ev > 0, ev, ev * 0.2)
                exv = jnp.exp(ev)
                hbuf[e, pl.ds(64, 16)] = exv
                for h in range(nheads):
                    sv = _bcast_lane(exv, h)
                    for j in range(bph):
                        cb = h * bph + j
                        hv = hbuf[e, pl.ds(cb * 16, 16)]
                        hbuf[e, pl.ds(cb * 16, 16)] = hv * sv
                return ecarry

            lax.fori_loop(0, _K, edge_body, None, unroll=5)

        for p in range(nph):
            htab = htabs[p]
            adq = adqs[0 if shared_adq else p]

            def issue_gather(i, b):
                pltpu.async_copy(htab.at[src_all.at[i]],
                                 hbufs[b].at[pl.ds(0, _K)], ghsems[b])
                pltpu.async_copy(adq.at[dst_all.at[i]], abufs[b], gasems[b])

            def wait_gather(b):
                pltpu.make_async_copy(htab.at[src_all.at[0]],
                                      hbufs[b].at[pl.ds(0, _K)],
                                      ghsems[b]).wait()
                pltpu.make_async_copy(adq.at[dst_all.at[0]], abufs[b],
                                      gasems[b]).wait()

            def issue_scatter(i, b):
                pltpu.async_copy(hbufs[b].at[pl.ds(0, _K)],
                                 acc.at[dst_all.at[i]], ssems[b], add=True)

            def wait_scatter(b):
                pltpu.make_async_copy(hbufs[b].at[pl.ds(0, _K)],
                                      acc.at[dst_all.at[0]], ssems[b]).wait()

            def zbody(r, carry):
                for cb in range(_AW // 16):
                    hrow0[r, pl.ds(cb * 16, 16)] = zero
                return carry

            lax.fori_loop(0, 128, zbody, None)
            for j in range(_ZR // 128):
                pltpu.sync_copy(hrow0.at[pl.ds(0, 128)],
                                acc.at[pl.ds(r0 + j * 128, 128)])
            plsc.subcore_barrier()

            issue_gather(0, 0)

            def pair_body(j, carry):
                a = 2 * j
                wait_gather(0)

                @pl.when(j > 0)
                def _():
                    wait_scatter(1)

                issue_gather(a + 1, 1)
                compute_chunk(hrow0, adrow0)
                issue_scatter(a, 0)
                wait_gather(1)

                @pl.when(j < _NPAIR - 1)
                def _():
                    wait_scatter(0)
                    issue_gather(a + 2, 0)

                compute_chunk(hrow1, adrow1)
                issue_scatter(a + 1, 1)
                return carry

            lax.fori_loop(0, _NPAIR, pair_body, None)
            wait_scatter(0)
            wait_scatter(1)
            plsc.subcore_barrier()

            obase = (p * 2 + cid) * _NP + r0
            for j in range(_ZR // 128):
                pltpu.sync_copy(acc.at[pl.ds(r0 + j * 128, 128)],
                                out.at[pl.ds(obase + j * 128, 128)])

    return kern


_sc_l1 = _make_sc_layer(4, 2, 2, False)
_sc_l2 = _make_sc_layer(2, 4, 1, False)
_sc_l3 = _make_sc_layer(2, 1, 4, True)


def _a1_body(x_ref, w1_ref, asm_ref, adm_ref,
             ht0, ht1, ht2, ht3, ad0, ad1, ad2, ad3, exs_ref):
    h = jnp.dot(x_ref[...], w1_ref[...], preferred_element_type=_f32)
    als = jnp.dot(h, asm_ref[...], preferred_element_type=_f32)
    ald = jnp.dot(h, adm_ref[...], preferred_element_type=_f32)
    z14 = jnp.zeros((_BN, 14), _f32)
    for p, ht in enumerate((ht0, ht1, ht2, ht3)):
        ht[...] = jnp.concatenate(
            [h[:, 64 * p:64 * p + 64], als[:, 2 * p:2 * p + 2], z14], axis=1)
    for p, ad in enumerate((ad0, ad1, ad2, ad3)):
        ad[...] = jnp.concatenate([ald[:, 2 * p:2 * p + 2], z14], axis=1)
    es = als + ald
    es = jnp.where(es > 0, es, es * 0.2)
    exs_ref[...] = jnp.exp(es)


def _n1_body(acc_ref, ht0_ref, ht1_ref, ht2_ref, ht3_ref, exs_ref, b1_ref,
             w2_ref, as2_ref, ad2_ref, r1_ref,
             o_ht0, o_ht1, o_ad0, o_ad1, exs2_ref):
    a = [acc_ref[2 * p] + acc_ref[2 * p + 1] for p in range(4)]
    exs = exs_ref[...]
    h1 = jnp.concatenate(
        [r[:, :64] for r in (ht0_ref, ht1_ref, ht2_ref, ht3_ref)], axis=1)
    num = jnp.concatenate([ap[:, :64] for ap in a], axis=1)
    den8 = jnp.concatenate([ap[:, 64:66] for ap in a], axis=1) + exs
    exb = jnp.dot(exs, r1_ref[...], preferred_element_type=_f32)
    denb = jnp.dot(den8, r1_ref[...], preferred_element_type=_f32)
    out1 = (num + exb * h1) / (denb + 1e-16) + b1_ref[...]
    h1r = jnp.maximum(out1, 0.0)
    h2 = jnp.dot(h1r, w2_ref[...], preferred_element_type=_f32)
    as2 = jnp.dot(h2, as2_ref[...], preferred_element_type=_f32)
    ad2 = jnp.dot(h2, ad2_ref[...], preferred_element_type=_f32)
    z12 = jnp.zeros((_BN, 12), _f32)
    for p, o in enumerate((o_ht0, o_ht1)):
        o[...] = jnp.concatenate(
            [h2[:, 64 * p:64 * p + 64], as2[:, 4 * p:4 * p + 4], z12], axis=1)
    for p, o in enumerate((o_ad0, o_ad1)):
        o[...] = jnp.concatenate([ad2[:, 4 * p:4 * p + 4], z12], axis=1)
    es = as2 + ad2
    es = jnp.where(es > 0, es, es * 0.2)
    exs2_ref[...] = jnp.exp(es)


def _n2_body(acc_ref, ht0_ref, ht1_ref, exs2_ref, b2_ref, w3_ref,
             as3_ref, ad3_ref, r2_ref,
             o_ht0, o_ht1, o_adq, exs3_ref):
    a = [acc_ref[2 * p] + acc_ref[2 * p + 1] for p in range(2)]
    exs = exs2_ref[...]
    h2 = jnp.concatenate([ht0_ref[:, :64], ht1_ref[:, :64]], axis=1)
    num = jnp.concatenate([ap[:, :64] for ap in a], axis=1)
    den8 = jnp.concatenate([ap[:, 64:68] for ap in a], axis=1) + exs
    exb = jnp.dot(exs, r2_ref[...], preferred_element_type=_f32)
    denb = jnp.dot(den8, r2_ref[...], preferred_element_type=_f32)
    out2 = (num + exb * h2) / (denb + 1e-16) + b2_ref[...]
    h2r = jnp.maximum(out2, 0.0)
    h3 = jnp.dot(h2r, w3_ref[...], preferred_element_type=_f32)
    as3 = jnp.sum(h3 * as3_ref[...], axis=1, keepdims=True)
    ad3 = jnp.sum(h3 * ad3_ref[...], axis=1, keepdims=True)
    z15 = jnp.zeros((_BN, 15), _f32)
    for p, o in enumerate((o_ht0, o_ht1)):
        o[...] = jnp.concatenate(
            [h3[:, 64 * p:64 * p + 64], as3, z15], axis=1)
    o_adq[...] = jnp.concatenate([ad3, z15], axis=1)
    es = as3 + ad3
    es = jnp.where(es > 0, es, es * 0.2)
    exs3_ref[...] = jnp.concatenate([jnp.exp(es), jnp.zeros((_BN, 7), _f32)],
                                    axis=1)


def _n3_body(acc_ref, ht0_ref, ht1_ref, exs3_ref, b3_ref, out_ref):
    a0 = acc_ref[0] + acc_ref[1]
    a1 = acc_ref[2] + acc_ref[3]
    ex3 = exs3_ref[:, 0:1]
    h3 = jnp.concatenate([ht0_ref[:, :64], ht1_ref[:, :64]], axis=1)
    num = jnp.concatenate([a0[:, :64], a1[:, :64]], axis=1)
    den = a0[:, 64:65] + ex3 + 1e-16
    out_ref[...] = (num + ex3 * h3) / den + b3_ref[...]


def _row_block(i):
    return (i, 0)


def _full_block(i):
    return (0, 0)


def _acc_block(i):
    return (0, i, 0)


def _ht_spec():
    return pl.BlockSpec((_BN, _AW), _row_block)


def _adq_spec():
    return pl.BlockSpec((_BN, 16), _row_block)


def _ht_shape():
    return jax.ShapeDtypeStruct((_N, _AW), _f32)


def _adq_shape():
    return jax.ShapeDtypeStruct((_N, 16), _f32)


def kernel(x, edge_index, W1, a_src1, a_dst1, b1, W2, a_src2, a_dst2, b2,
           W3, a_src3, a_dst3, b3):
    src = edge_index[0].reshape(_NW, _NCH, _K)
    dst = edge_index[1].reshape(_NW, _NCH, _K)
    eye8 = jnp.eye(8, dtype=_f32)
    AS1 = jnp.einsum("hk,hc->hck", eye8, a_src1).reshape(256, 8)
    AD1 = jnp.einsum("hk,hc->hck", eye8, a_dst1).reshape(256, 8)
    AS2 = jnp.einsum("hk,hc->hck", eye8, a_src2).reshape(128, 8)
    AD2 = jnp.einsum("hk,hc->hck", eye8, a_dst2).reshape(128, 8)
    R1 = jnp.kron(eye8, jnp.ones((1, 32), _f32))
    R2 = jnp.kron(eye8, jnp.ones((1, 16), _f32))

    l1 = pl.pallas_call(
        _a1_body,
        grid=(_GRID,),
        in_specs=[
            pl.BlockSpec((_BN, 128), _row_block),
            pl.BlockSpec((128, 256), _full_block),
            pl.BlockSpec((256, 8), _full_block),
            pl.BlockSpec((256, 8), _full_block),
        ],
        out_specs=[_ht_spec() for _ in range(4)]
        + [_adq_spec() for _ in range(4)]
        + [pl.BlockSpec((_BN, 8), _row_block)],
        out_shape=[_ht_shape() for _ in range(4)]
        + [_adq_shape() for _ in range(4)]
        + [jax.ShapeDtypeStruct((_N, 8), _f32)],
    )(x, W1, AS1, AD1)
    hts1, adqs1, exs1 = l1[:4], l1[4:8], l1[8]

    acc1 = _sc_l1(*hts1, *adqs1, src, dst).reshape(8, _NP, _AW)

    ht2_0, ht2_1, ad2_0, ad2_1, exs2 = pl.pallas_call(
        _n1_body,
        grid=(_GRID,),
        in_specs=[
            pl.BlockSpec((8, _BN, _AW), _acc_block),
            _ht_spec(), _ht_spec(), _ht_spec(), _ht_spec(),
            pl.BlockSpec((_BN, 8), _row_block),
            pl.BlockSpec((1, 256), _full_block),
            pl.BlockSpec((256, 128), _full_block),
            pl.BlockSpec((128, 8), _full_block),
            pl.BlockSpec((128, 8), _full_block),
            pl.BlockSpec((8, 256), _full_block),
        ],
        out_specs=[_ht_spec(), _ht_spec(), _adq_spec(), _adq_spec(),
                   pl.BlockSpec((_BN, 8), _row_block)],
        out_shape=[_ht_shape(), _ht_shape(), _adq_shape(), _adq_shape(),
                   jax.ShapeDtypeStruct((_N, 8), _f32)],
    )(acc1, *hts1, exs1, b1.reshape(1, 256), W2, AS2, AD2, R1)

    acc2 = _sc_l2(ht2_0, ht2_1, ad2_0, ad2_1, src, dst).reshape(4, _NP, _AW)

    ht3_0, ht3_1, adq3, exs3 = pl.pallas_call(
        _n2_body,
        grid=(_GRID,),
        in_specs=[
            pl.BlockSpec((4, _BN, _AW), _acc_block),
            _ht_spec(), _ht_spec(),
            pl.BlockSpec((_BN, 8), _row_block),
            pl.BlockSpec((1, 128), _full_block),
            pl.BlockSpec((128, 128), _full_block),
            pl.BlockSpec((1, 128), _full_block),
            pl.BlockSpec((1, 128), _full_block),
            pl.BlockSpec((8, 128), _full_block),
        ],
        out_specs=[_ht_spec(), _ht_spec(), _adq_spec(),
                   pl.BlockSpec((_BN, 8), _row_block)],
        out_shape=[_ht_shape(), _ht_shape(), _adq_shape(),
                   jax.ShapeDtypeStruct((_N, 8), _f32)],
    )(acc2, ht2_0, ht2_1, exs2, b2.reshape(1, 128), W3,
      a_src3.reshape(1, 128), a_dst3.reshape(1, 128), R2)

    acc3 = _sc_l3(ht3_0, ht3_1, adq3, src, dst).reshape(4, _NP, _AW)

    out = pl.pallas_call(
        _n3_body,
        grid=(_GRID,),
        in_specs=[
            pl.BlockSpec((4, _BN, _AW), _acc_block),
            _ht_spec(), _ht_spec(),
            pl.BlockSpec((_BN, 8), _row_block),
            pl.BlockSpec((1, 128), _full_block),
        ],
        out_specs=pl.BlockSpec((_BN, 128), _row_block),
        out_shape=jax.ShapeDtypeStruct((_N, 128), _f32),
    )(acc3, ht3_0, ht3_1, exs3, b3.reshape(1, 128))

    return out
